# baseline (speedup 1.0000x reference)
import functools

import jax
import jax.numpy as jnp
from jax import lax
from jax.experimental import pallas as pl
from jax.experimental.pallas import tpu as pltpu
from jax.experimental.pallas import tpu_sc as plsc

_CHUNK = 400
_NBUF = 2


@functools.lru_cache(maxsize=None)
def _build_gather(n: int, vocab: int, dim: int):
    info = plsc.get_sparse_core_info()
    nc, ns, lanes = info.num_cores, info.num_subcores, info.num_lanes
    nw = nc * ns
    assert dim % lanes == 0 and n % (8 * nw) == 0
    n_per_w = n // nw
    chunk = _CHUNK
    nbuf = _NBUF
    assert n_per_w % chunk == 0 and chunk % 8 == 0
    rounds = n_per_w // chunk

    mesh = plsc.VectorSubcoreMesh(core_axis_name="c", subcore_axis_name="s")

    @functools.partial(
        pl.kernel,
        mesh=mesh,
        out_type=jax.ShapeDtypeStruct((n, 128), jnp.float32),
        scratch_types=[
            pltpu.VMEM((n_per_w,), jnp.int32),
            pltpu.VMEM((nbuf, chunk, 128), jnp.float32),
        ]
        + [pltpu.SemaphoreType.DMA] * (2 * nbuf),
    )
    def gather_kernel(table_hbm, idx_hbm, out_hbm, idx_v, bufs, *sems):
        sem_g = sems[:nbuf]
        sem_s = sems[nbuf:]
        wid = lax.axis_index("s") * nc + lax.axis_index("c")
        base = wid * n_per_w
        pltpu.sync_copy(idx_hbm.at[pl.ds(base, n_per_w)], idx_v)

        def gather_start(r, slot):
            pltpu.async_copy(
                table_hbm.at[idx_v.at[pl.ds(r * chunk, chunk)]],
                bufs.at[slot],
                sem_g[slot],
            )

        def gather_wait(r, slot):
            pltpu.make_async_copy(
                table_hbm.at[idx_v.at[pl.ds(r * chunk, chunk)]],
                bufs.at[slot],
                sem_g[slot],
            ).wait()

        def store_start(r, slot):
            return pltpu.async_copy(
                bufs.at[slot],
                out_hbm.at[pl.ds(base + r * chunk, chunk)],
                sem_s[slot],
            )

        for slot in range(nbuf):
            gather_start(slot, slot)

        def step(r, carry):
            stores = []
            for slot in range(nbuf):
                gather_wait(r * nbuf + slot, slot)
                stores.append(store_start(r * nbuf + slot, slot))
            for slot in range(nbuf):
                stores[slot].wait()
                gather_start(r * nbuf + slot + nbuf, slot)
            return carry

        lax.fori_loop(0, rounds // nbuf - 1, step, 0)

        stores = []
        for slot in range(nbuf):
            r = rounds - nbuf + slot
            gather_wait(r, slot)
            stores.append(store_start(r, slot))
        for h in stores:
            h.wait()

    return gather_kernel


def kernel(x, emb_wi):
    b, s = x.shape
    v, d = emb_wi.shape
    run = _build_gather(b * s, v, d)
    table_p = jnp.pad(emb_wi, ((0, 0), (0, 128 - d)))
    out = run(table_p, x.reshape(-1).astype(jnp.int32))
    return out[:, :d].reshape(b, s, d)


# speedup vs baseline: 1.5152x; 1.5152x over previous
import functools

import jax
import jax.numpy as jnp
from jax import lax
from jax.experimental import pallas as pl
from jax.experimental.pallas import tpu as pltpu
from jax.experimental.pallas import tpu_sc as plsc

_CHUNK_B = 8
_NBUF = 2


@functools.lru_cache(maxsize=None)
def _build_gather(batch: int, seq: int, vocab: int, dim: int):
    info = plsc.get_sparse_core_info()
    nc, ns, lanes = info.num_cores, info.num_subcores, info.num_lanes
    nw = nc * ns
    assert dim % lanes == 0 and batch % nw == 0
    b_per_w = batch // nw
    cb = _CHUNK_B
    chunk = cb * seq
    nbuf = _NBUF
    assert b_per_w % (cb * nbuf) == 0 and chunk % 8 == 0
    rounds = b_per_w // cb

    mesh = plsc.VectorSubcoreMesh(core_axis_name="c", subcore_axis_name="s")

    @functools.partial(
        pl.kernel,
        mesh=mesh,
        out_type=jax.ShapeDtypeStruct((batch, seq, 128), jnp.float32),
        scratch_types=[
            pltpu.VMEM((b_per_w * seq,), jnp.int32),
            pltpu.VMEM((nbuf, chunk, 128), jnp.float32),
        ]
        + [pltpu.SemaphoreType.DMA] * (2 * nbuf),
    )
    def gather_kernel(table_hbm, idx_hbm, out_hbm, idx_v, bufs, *sems):
        sem_g = sems[:nbuf]
        sem_s = sems[nbuf:]
        wid = lax.axis_index("s") * nc + lax.axis_index("c")
        b0 = wid * b_per_w
        pltpu.sync_copy(idx_hbm.at[pl.ds(b0 * seq, b_per_w * seq)], idx_v)

        def gather_start(r, slot):
            pltpu.async_copy(
                table_hbm.at[idx_v.at[pl.ds(r * chunk, chunk)]],
                bufs.at[slot],
                sem_g[slot],
            )

        def gather_wait(r, slot):
            pltpu.make_async_copy(
                table_hbm.at[idx_v.at[pl.ds(r * chunk, chunk)]],
                bufs.at[slot],
                sem_g[slot],
            ).wait()

        def store_start(r, slot):
            handles = []
            for k in range(cb):
                handles.append(
                    pltpu.async_copy(
                        bufs.at[slot, pl.ds(k * seq, seq)],
                        out_hbm.at[b0 + r * cb + k],
                        sem_s[slot],
                    )
                )
            return handles

        for slot in range(nbuf):
            gather_start(slot, slot)

        def step(r, carry):
            stores = []
            for slot in range(nbuf):
                gather_wait(r * nbuf + slot, slot)
                stores.append(store_start(r * nbuf + slot, slot))
            for slot in range(nbuf):
                for h in stores[slot]:
                    h.wait()
                gather_start(r * nbuf + slot + nbuf, slot)
            return carry

        lax.fori_loop(0, rounds // nbuf - 1, step, 0)

        stores = []
        for slot in range(nbuf):
            r = rounds - nbuf + slot
            gather_wait(r, slot)
            stores.append(store_start(r, slot))
        for hs in stores:
            for h in hs:
                h.wait()

    return gather_kernel


def kernel(x, emb_wi):
    b, s = x.shape
    v, d = emb_wi.shape
    run = _build_gather(b, s, v, d)
    table_p = jnp.pad(emb_wi, ((0, 0), (0, 128 - d)))
    out = run(table_p, x.reshape(-1).astype(jnp.int32))
    return out[:, :, :d]


# speedup vs baseline: 1.5557x; 1.0267x over previous
import functools

import jax
import jax.numpy as jnp
from jax import lax
from jax.experimental import pallas as pl
from jax.experimental.pallas import tpu as pltpu
from jax.experimental.pallas import tpu_sc as plsc

_NBUF = 8


@functools.lru_cache(maxsize=None)
def _build_gather(batch: int, seq: int, vocab: int, dim: int):
    info = plsc.get_sparse_core_info()
    nc, ns, lanes = info.num_cores, info.num_subcores, info.num_lanes
    nw = nc * ns
    assert dim % lanes == 0 and batch % nw == 0
    b_per_w = batch // nw
    nbuf = _NBUF
    assert b_per_w % nbuf == 0
    rounds = b_per_w

    mesh = plsc.VectorSubcoreMesh(core_axis_name="c", subcore_axis_name="s")

    @functools.partial(
        pl.kernel,
        mesh=mesh,
        out_type=jax.ShapeDtypeStruct((batch, seq, 128), jnp.float32),
        scratch_types=[
            pltpu.VMEM((b_per_w, seq), jnp.int32),
            pltpu.VMEM((nbuf, seq, 128), jnp.float32),
        ]
        + [pltpu.SemaphoreType.DMA] * (2 * _NBUF),
    )
    def gather_kernel(table_hbm, x_hbm, out_hbm, idx_v, bufs, *sems):
        sem_g = sems[:nbuf]
        sem_s = sems[nbuf:]
        wid = lax.axis_index("s") * nc + lax.axis_index("c")
        b0 = wid * b_per_w
        pltpu.sync_copy(x_hbm.at[pl.ds(b0, b_per_w)], idx_v)

        def gather_start(r, slot):
            pltpu.async_copy(
                table_hbm.at[idx_v.at[r, pl.ds(0, seq)]],
                bufs.at[slot],
                sem_g[slot],
            )

        def gather_wait(r, slot):
            pltpu.make_async_copy(
                table_hbm.at[idx_v.at[r, pl.ds(0, seq)]],
                bufs.at[slot],
                sem_g[slot],
            ).wait()

        def store_start(r, slot):
            return pltpu.async_copy(
                bufs.at[slot],
                out_hbm.at[b0 + r],
                sem_s[slot],
            )

        for slot in range(nbuf):
            gather_start(slot, slot)

        def step(g, carry):
            stores = []
            for slot in range(nbuf):
                gather_wait(g * nbuf + slot, slot)
                stores.append(store_start(g * nbuf + slot, slot))
            for slot in range(nbuf):
                stores[slot].wait()
                gather_start(g * nbuf + slot + nbuf, slot)
            return carry

        lax.fori_loop(0, rounds // nbuf - 1, step, 0)

        stores = []
        for slot in range(nbuf):
            r = rounds - nbuf + slot
            gather_wait(r, slot)
            stores.append(store_start(r, slot))
        for h in stores:
            h.wait()

    return gather_kernel


def kernel(x, emb_wi):
    b, s = x.shape
    v, d = emb_wi.shape
    run = _build_gather(b, s, v, d)
    table_p = jnp.pad(emb_wi, ((0, 0), (0, 128 - d)))
    out = run(table_p, x.astype(jnp.int32))
    return out[:, :, :d]
